# Initial kernel scaffold; baseline (speedup 1.0000x reference)
import jax
import jax.numpy as jnp
from jax.experimental import pallas as pl


def kernel(x, edge_index, edge_attr, masks, complement_masks, W1, b1, W2, b2, eps, gamma, beta):
    raise NotImplementedError("write your pallas kernel here")


# speedup vs baseline: 3.5644x; 3.5644x over previous
import functools

import jax
import jax.numpy as jnp
from jax import lax
from jax.experimental import pallas as pl
from jax.experimental.pallas import tpu as pltpu
from jax.experimental.pallas import tpu_sc as plsc

N = 10000
E = 320000
D = 128
C = 4
BN_EPS = 1e-5

NC = 2
NS = 16
NW = NC * NS
EPT = E // NW
K = 80
NCHUNK = EPT // K
RPT = 624
REM = N - NS * RPT
ZR = 104
LANES = 16
G = D // LANES


def _sc_edge_body(x_hbm, src_hbm, dst_hbm, ea_hbm, out_hbm,
                  aggr_sh, src_v, dst_v, xbuf, ebuf, zbuf, sem):
    c = lax.axis_index("c")
    s = lax.axis_index("s")
    wid = c * NS + s

    def zrow(r, carry):
        for g in range(G):
            zbuf[r, pl.ds(LANES * g, LANES)] = jnp.zeros((LANES,), jnp.float32)
        return carry
    lax.fori_loop(0, ZR, zrow, 0)
    for j in range(RPT // ZR):
        pltpu.sync_copy(zbuf, aggr_sh.at[pl.ds(s * RPT + j * ZR, ZR)])

    @pl.when(s == NS - 1)
    def _zero_rem():
        pltpu.sync_copy(zbuf.at[pl.ds(0, REM)], aggr_sh.at[pl.ds(NS * RPT, REM)])
    plsc.subcore_barrier()

    tile_base = wid * EPT

    def chunk(i, carry):
        base = tile_base + i * K
        pltpu.sync_copy(src_hbm.at[pl.ds(base, K)], src_v)
        pltpu.sync_copy(dst_hbm.at[pl.ds(base, K)], dst_v)
        gat = pltpu.async_copy(x_hbm.at[src_v], xbuf, sem)
        pltpu.sync_copy(ea_hbm.at[pl.ds(base, K)], ebuf)
        gat.wait()

        def row(r, rcarry):
            for g in range(G):
                sl = pl.ds(LANES * g, LANES)
                ebuf[r, sl] = jnp.maximum(xbuf[r, sl] + ebuf[r, sl], 0.0)
            return rcarry
        lax.fori_loop(0, K, row, 0)

        pltpu.sync_copy(ebuf, aggr_sh.at[dst_v], add=True)
        return carry
    lax.fori_loop(0, NCHUNK, chunk, 0)

    plsc.subcore_barrier()
    pltpu.sync_copy(aggr_sh.at[pl.ds(s * RPT, RPT)],
                    out_hbm.at[c, pl.ds(s * RPT, RPT)])

    @pl.when(s == NS - 1)
    def _copy_rem():
        pltpu.sync_copy(aggr_sh.at[pl.ds(NS * RPT, REM)],
                        out_hbm.at[c, pl.ds(NS * RPT, REM)])


_sc_edge = functools.partial(
    pl.kernel,
    mesh=plsc.VectorSubcoreMesh(core_axis_name="c", subcore_axis_name="s"),
    out_type=jax.ShapeDtypeStruct((NC, N, D), jnp.float32),
    scratch_types=[
        pltpu.VMEM_SHARED((N, D), jnp.float32),
        pltpu.VMEM((K,), jnp.int32),
        pltpu.VMEM((K,), jnp.int32),
        pltpu.VMEM((K, D), jnp.float32),
        pltpu.VMEM((K, D), jnp.float32),
        pltpu.VMEM((ZR, D), jnp.float32),
        pltpu.SemaphoreType.DMA,
    ],
)(_sc_edge_body)


def _tc_body(final, x_ref, aggr_ref, w1_ref, b1_ref, w2_ref, b2_ref,
             mask_ref, gamma_ref, beta_ref, xin_ref, eps_ref, out_ref):
    x = x_ref[...]
    a = aggr_ref[0] + aggr_ref[1]
    h = (1.0 + eps_ref[0, 0]) * x + a
    h = jnp.maximum(jnp.dot(h, w1_ref[...],
                            preferred_element_type=jnp.float32) + b1_ref[...], 0.0)
    h = jnp.dot(h, w2_ref[...], preferred_element_type=jnp.float32) + b2_ref[...]
    y = mask_ref[...] * h + x
    mu = jnp.mean(y, axis=0, keepdims=True)
    var = jnp.mean((y - mu) * (y - mu), axis=0, keepdims=True)
    y = gamma_ref[...] * (y - mu) * lax.rsqrt(var + BN_EPS) + beta_ref[...]
    if final:
        y = xin_ref[...] + jnp.maximum(y, 0.0)
    out_ref[...] = y


def _tc_update(x, aggr2, w1, b1, w2, b2, mask, gamma, beta, x_in, eps_c, final):
    return pl.pallas_call(
        functools.partial(_tc_body, final),
        out_shape=jax.ShapeDtypeStruct((N, D), jnp.float32),
        in_specs=[pl.BlockSpec(memory_space=pltpu.VMEM)] * 10
        + [pl.BlockSpec(memory_space=pltpu.SMEM)],
    )(x, aggr2, w1, b1, w2, b2, mask, gamma, beta, x_in, eps_c)


def kernel(x, edge_index, edge_attr, masks, complement_masks,
           W1, b1, W2, b2, eps, gamma, beta):
    src = edge_index[0]
    dst = edge_index[1]
    x_in = x
    for c in range(C):
        aggr2 = _sc_edge(x, src, dst, edge_attr)
        x = _tc_update(
            x, aggr2, W1[c], b1[c].reshape(1, D), W2[c], b2[c].reshape(1, D),
            masks[c].reshape(N, 1), gamma[c].reshape(1, D), beta[c].reshape(1, D),
            x_in, eps[c].reshape(1, 1), final=(c == C - 1))
    return x
